# baseline (speedup 1.0000x reference)
import jax
import jax.numpy as jnp
from jax.experimental import pallas as pl
from jax.experimental.pallas import tpu as pltpu

N_LEADS = 12
D_MODEL = 256
BATCH = 16384
CHUNK = 256
NCHUNK = BATCH // CHUNK
NBUF = 4


def _gather_body(pos_ref, w_ref, o_ref):
    for i in range(N_LEADS):
        o_ref[i, :] = w_ref[pos_ref[0, i], :]


def _add_body(enc_ref, x_ref, o_ref, in_buf, out_buf, in_sem, out_sem):
    enc = enc_ref[...][None, :, :]

    def start_in(c, b):
        pltpu.make_async_copy(
            x_ref.at[pl.ds(c * CHUNK, CHUNK)], in_buf.at[b], in_sem.at[b]
        ).start()

    for b in range(NBUF):
        start_in(b, b)

    for c in range(NCHUNK):
        b = c % NBUF
        pltpu.make_async_copy(
            x_ref.at[pl.ds(c * CHUNK, CHUNK)], in_buf.at[b], in_sem.at[b]
        ).wait()
        if c >= NBUF:
            pltpu.make_async_copy(
                out_buf.at[b], o_ref.at[pl.ds((c - NBUF) * CHUNK, CHUNK)],
                out_sem.at[b],
            ).wait()
        out_buf[b] = in_buf[b] + enc
        pltpu.make_async_copy(
            out_buf.at[b], o_ref.at[pl.ds(c * CHUNK, CHUNK)], out_sem.at[b]
        ).start()
        if c + NBUF < NCHUNK:
            start_in(c + NBUF, b)

    for c in range(NCHUNK - NBUF, NCHUNK):
        b = c % NBUF
        pltpu.make_async_copy(
            out_buf.at[b], o_ref.at[pl.ds(c * CHUNK, CHUNK)], out_sem.at[b]
        ).wait()


def kernel(x, encoding_weight, positions):
    pos2d = positions.astype(jnp.int32).reshape(1, N_LEADS)
    pos_enc = pl.pallas_call(
        _gather_body,
        in_specs=[
            pl.BlockSpec(memory_space=pltpu.SMEM),
            pl.BlockSpec(memory_space=pltpu.VMEM),
        ],
        out_shape=jax.ShapeDtypeStruct((N_LEADS, D_MODEL), jnp.float32),
    )(pos2d, encoding_weight)

    return pl.pallas_call(
        _add_body,
        in_specs=[
            pl.BlockSpec(memory_space=pltpu.VMEM),
            pl.BlockSpec(memory_space=pl.ANY),
        ],
        out_specs=pl.BlockSpec(memory_space=pl.ANY),
        out_shape=jax.ShapeDtypeStruct((BATCH, N_LEADS, D_MODEL), jnp.float32),
        scratch_shapes=[
            pltpu.VMEM((NBUF, CHUNK, N_LEADS, D_MODEL), jnp.float32),
            pltpu.VMEM((NBUF, CHUNK, N_LEADS, D_MODEL), jnp.float32),
            pltpu.SemaphoreType.DMA((NBUF,)),
            pltpu.SemaphoreType.DMA((NBUF,)),
        ],
    )(pos_enc, x)


# speedup vs baseline: 3.7193x; 3.7193x over previous
import jax
import jax.numpy as jnp
from jax.experimental import pallas as pl
from jax.experimental.pallas import tpu as pltpu

N_LEADS = 12
D_MODEL = 256
BATCH = 16384
BLOCK_B = 2048


def _add_body(pos_ref, w_ref, x_ref, o_ref):
    lead = pl.program_id(0)
    row = pos_ref[0, lead]
    enc = w_ref[row, :]
    o_ref[...] = x_ref[...] + enc[None, None, :]


def kernel(x, encoding_weight, positions):
    pos2d = positions.astype(jnp.int32).reshape(1, N_LEADS)
    x_t = jnp.transpose(x, (1, 0, 2))
    out_t = pl.pallas_call(
        _add_body,
        grid=(N_LEADS, BATCH // BLOCK_B),
        in_specs=[
            pl.BlockSpec(memory_space=pltpu.SMEM),
            pl.BlockSpec((N_LEADS, D_MODEL), lambda l, i: (0, 0)),
            pl.BlockSpec((1, BLOCK_B, D_MODEL), lambda l, i: (l, i, 0)),
        ],
        out_specs=pl.BlockSpec((1, BLOCK_B, D_MODEL), lambda l, i: (l, i, 0)),
        out_shape=jax.ShapeDtypeStruct((N_LEADS, BATCH, D_MODEL), jnp.float32),
    )(pos2d, encoding_weight, x_t)
    return jnp.transpose(out_t, (1, 0, 2))


# speedup vs baseline: 4.1271x; 1.1097x over previous
import jax
import jax.numpy as jnp
from jax.experimental import pallas as pl
from jax.experimental.pallas import tpu as pltpu

N_LEADS = 12
D_MODEL = 256
BATCH = 16384
BLOCK_B = 8192


def _add_body(pos_ref, w_ref, x_ref, o_ref):
    lead = pl.program_id(0)
    row = pos_ref[0, lead]
    enc = w_ref[row, :]
    o_ref[...] = x_ref[...] + enc[None, None, :]


def kernel(x, encoding_weight, positions):
    pos2d = positions.astype(jnp.int32).reshape(1, N_LEADS)
    x_t = jnp.transpose(x, (1, 0, 2))
    out_t = pl.pallas_call(
        _add_body,
        grid=(N_LEADS, BATCH // BLOCK_B),
        in_specs=[
            pl.BlockSpec(memory_space=pltpu.SMEM),
            pl.BlockSpec((N_LEADS, D_MODEL), lambda l, i: (0, 0)),
            pl.BlockSpec((1, BLOCK_B, D_MODEL), lambda l, i: (l, i, 0)),
        ],
        out_specs=pl.BlockSpec((1, BLOCK_B, D_MODEL), lambda l, i: (l, i, 0)),
        out_shape=jax.ShapeDtypeStruct((N_LEADS, BATCH, D_MODEL), jnp.float32),
    )(pos2d, encoding_weight, x_t)
    return jnp.transpose(out_t, (1, 0, 2))
